# baseline (speedup 1.0000x reference)
import dataclasses
import jax
import jax.numpy as jnp
from jax import lax
from jax.experimental import pallas as pl
from jax.experimental.pallas import tpu as pltpu
from jax.experimental.pallas import tpu_sc as plsc

N = 10000
D = 128
H = 8
C = 16
NP = 10240
NS = NP // 16
E = 320000
NW = 32
B = 32
EPW = 10112
EP = NW * EPW
EPA = EP + 2 * B
NB = EPW // B
RPT = NP // 16
SPT = NS // 16

_f32 = jnp.float32
_i32 = jnp.int32


def _proj_body(x_ref, wl_ref, bl_ref, wr_ref, br_ref, xl_ref, xr_ref):
    xb = x_ref[...]
    xl_ref[...] = (
        jnp.dot(xb, wl_ref[...], preferred_element_type=_f32) + bl_ref[...]
    )
    xr_ref[...] = (
        jnp.dot(xb, wr_ref[...], preferred_element_type=_f32) + br_ref[...]
    )


def _proj(x, Wl, bl, Wr, br):
    BR = 256
    return pl.pallas_call(
        _proj_body,
        grid=(NP // BR,),
        in_specs=[
            pl.BlockSpec((BR, D), lambda i: (i, 0)),
            pl.BlockSpec((D, D), lambda i: (0, 0)),
            pl.BlockSpec((1, D), lambda i: (0, 0)),
            pl.BlockSpec((D, D), lambda i: (0, 0)),
            pl.BlockSpec((1, D), lambda i: (0, 0)),
        ],
        out_specs=[
            pl.BlockSpec((BR, D), lambda i: (i, 0)),
            pl.BlockSpec((BR, D), lambda i: (i, 0)),
        ],
        out_shape=[
            jax.ShapeDtypeStruct((NP, D), _f32),
            jax.ShapeDtypeStruct((NP, D), _f32),
        ],
    )(x, Wl, bl.reshape(1, D), Wr, br.reshape(1, D))


def _edge_body(xl_hbm, xr_hbm, src_hbm, dst_hbm, att_hbm,
               out_hbm, souts_hbm,
               acc_sh, s_sh,
               srcv0, srcv1, dstv0, dstv1, dsts0, dsts1, dst160, dst161,
               xlj0, xlj1, xri0, xri1, msg0, msg1, exm0, exm1,
               attv, semg0, semg1, sems0, sems1, semi0, semi1):
    cid = lax.axis_index("c")
    sid = lax.axis_index("s")
    wid = cid * 16 + sid

    srcv = [srcv0, srcv1]
    dstv = [dstv0, dstv1]
    dsts = [dsts0, dsts1]
    dst16 = [dst160, dst161]
    xlj = [xlj0, xlj1]
    xri = [xri0, xri1]
    msg = [msg0, msg1]
    exm = [exm0, exm1]
    semg = [semg0, semg1]
    sems = [sems0, sems1]
    semi = [semi0, semi1]

    zv = jnp.zeros((C,), _f32)
    zvi = jnp.zeros((C,), _i32)
    iota16 = lax.iota(_i32, 16)
    zero16 = jnp.zeros((16,), _f32)

    for p in range(2):
        for r in range(B):
            for j in range(D // C):
                exm[p][r, pl.ds(j * C, C)] = zv
        for g in range(B // 16):
            dsts[p][pl.ds(g * 16, 16)] = zvi
            dst16[p][pl.ds(g * 16, 16)] = zvi

    @pl.loop(0, RPT // B)
    def _zero(k):
        pltpu.sync_copy(exm0, acc_sh.at[pl.ds(sid * RPT + k * B, B)])

    pltpu.sync_copy(exm0, s_sh.at[pl.ds(sid * SPT, B)])
    pltpu.sync_copy(exm0.at[pl.ds(0, SPT - B)],
                    s_sh.at[pl.ds(sid * SPT + B, SPT - B)])

    pltpu.sync_copy(att_hbm, attv)
    att_rows = [attv[h, :] for h in range(H)]

    base0 = wid * EPW
    pltpu.async_copy(src_hbm.at[pl.ds(base0, B)], srcv[0], semi[0])
    pltpu.async_copy(dst_hbm.at[pl.ds(base0, B)], dstv[0], semi[0])
    pltpu.async_copy(src_hbm.at[pl.ds(base0 + B, B)], srcv[1], semi[1])
    pltpu.async_copy(dst_hbm.at[pl.ds(base0 + B, B)], dstv[1], semi[1])
    pltpu.make_async_copy(src_hbm.at[pl.ds(base0, B)], srcv[0],
                          semi[0]).wait()
    pltpu.make_async_copy(dst_hbm.at[pl.ds(base0, B)], dstv[0],
                          semi[0]).wait()
    pltpu.async_copy(xl_hbm.at[srcv[0]], xlj[0], semg[0])
    pltpu.async_copy(xr_hbm.at[dstv[0]], xri[0], semg[0])
    for p in range(2):
        pltpu.async_copy(exm[p], acc_sh.at[dsts[p]], sems[p], add=True)
        pltpu.async_copy(exm[p], s_sh.at[dst16[p]], sems[p], add=True)

    plsc.subcore_barrier()

    @pl.loop(0, NB // 2)
    def _iter(bj):
        for p in range(2):
            bi = 2 * bj + p
            q = 1 - p
            pltpu.make_async_copy(xl_hbm.at[srcv[p]], xlj[p],
                                  semg[p]).wait()
            pltpu.make_async_copy(xr_hbm.at[dstv[p]], xri[p],
                                  semg[p]).wait()
            pltpu.make_async_copy(src_hbm.at[pl.ds(0, B)], srcv[q],
                                  semi[q]).wait()
            pltpu.make_async_copy(dst_hbm.at[pl.ds(0, B)], dstv[q],
                                  semi[q]).wait()
            pltpu.async_copy(xl_hbm.at[srcv[q]], xlj[q], semg[q])
            pltpu.async_copy(xr_hbm.at[dstv[q]], xri[q], semg[q])
            pltpu.make_async_copy(msg[p], acc_sh.at[dsts[p]],
                                  sems[p]).wait()
            pltpu.make_async_copy(exm[p], s_sh.at[dst16[p]],
                                  sems[p]).wait()
            for g in range(B // 16):
                dd = dsts[p][pl.ds(g * 16, 16)]
                cmb = (dd & 15) * 8
                rows = g * 16 + iota16
                for h in range(H):
                    plsc.store_scatter(exm[p], [rows, cmb + h], zero16)
            rows0 = iota16
            rows1 = 16 + iota16
            d0 = dstv[p][pl.ds(0, 16)]
            d1 = dstv[p][pl.ds(16, 16)]
            dsts[p][pl.ds(0, 16)] = d0
            dsts[p][pl.ds(16, 16)] = d1
            dst16[p][pl.ds(0, 16)] = lax.shift_right_logical(d0, 4)
            dst16[p][pl.ds(16, 16)] = lax.shift_right_logical(d1, 4)
            cmb0 = (d0 & 15) * 8
            cmb1 = (d1 & 15) * 8
            for h in range(H):
                def _logit(c, lgs, _h=h):
                    lg0, lg1 = lgs
                    dcol = (c + iota16) & 15
                    col = _h * C + dcol
                    a = jnp.take(att_rows[_h], dcol)
                    l0 = plsc.load_gather(xlj[p], [rows0, col])
                    r0 = plsc.load_gather(xri[p], [rows0, col])
                    l1 = plsc.load_gather(xlj[p], [rows1, col])
                    r1 = plsc.load_gather(xri[p], [rows1, col])
                    z0 = l0 + r0
                    z0 = jnp.maximum(z0, 0.2 * z0)
                    z1 = l1 + r1
                    z1 = jnp.maximum(z1, 0.2 * z1)
                    return (lg0 + z0 * a, lg1 + z1 * a)

                zz = jnp.zeros((16,), _f32)
                lg0, lg1 = lax.fori_loop(0, C, _logit, (zz, zz),
                                         unroll=8)
                ex0 = jnp.exp(lg0)
                ex1 = jnp.exp(lg1)
                plsc.store_scatter(exm[p], [rows0, cmb0 + h], ex0)
                plsc.store_scatter(exm[p], [rows1, cmb1 + h], ex1)

                def _msg(c, t, _h=h, _ex0=ex0, _ex1=ex1):
                    dcol = (c + iota16) & 15
                    col = _h * C + dcol
                    l0 = plsc.load_gather(xlj[p], [rows0, col])
                    l1 = plsc.load_gather(xlj[p], [rows1, col])
                    plsc.store_scatter(msg[p], [rows0, col], _ex0 * l0)
                    plsc.store_scatter(msg[p], [rows1, col], _ex1 * l1)
                    return t

                lax.fori_loop(0, C, _msg, 0, unroll=8)
            pltpu.async_copy(msg[p], acc_sh.at[dsts[p]], sems[p],
                             add=True)
            pltpu.async_copy(exm[p], s_sh.at[dst16[p]], sems[p],
                             add=True)
            base2 = wid * EPW + (bi + 2) * B
            pltpu.async_copy(src_hbm.at[pl.ds(base2, B)], srcv[p],
                             semi[p])
            pltpu.async_copy(dst_hbm.at[pl.ds(base2, B)], dstv[p],
                             semi[p])

    for p in range(2):
        pltpu.make_async_copy(msg[p], acc_sh.at[dsts[p]], sems[p]).wait()
        pltpu.make_async_copy(exm[p], s_sh.at[dst16[p]], sems[p]).wait()
    pltpu.make_async_copy(xl_hbm.at[srcv[0]], xlj[0], semg[0]).wait()
    pltpu.make_async_copy(xr_hbm.at[dstv[0]], xri[0], semg[0]).wait()
    pltpu.make_async_copy(src_hbm.at[pl.ds(0, B)], srcv[1],
                          semi[1]).wait()
    pltpu.make_async_copy(dst_hbm.at[pl.ds(0, B)], dstv[1],
                          semi[1]).wait()

    plsc.subcore_barrier()

    @pl.loop(0, RPT // B)
    def _out(k):
        r0 = sid * RPT + k * B
        pltpu.sync_copy(acc_sh.at[pl.ds(r0, B)],
                        out_hbm.at[cid, pl.ds(r0, B)])

    pltpu.sync_copy(s_sh.at[pl.ds(sid * SPT, B)],
                    souts_hbm.at[cid, pl.ds(sid * SPT, B)])
    pltpu.sync_copy(s_sh.at[pl.ds(sid * SPT + B, SPT - B)],
                    souts_hbm.at[cid, pl.ds(sid * SPT + B, SPT - B)])


def _edge_pass(xl, xr, src, dst, att):
    mesh = plsc.VectorSubcoreMesh(core_axis_name="c", subcore_axis_name="s")
    cp = pltpu.CompilerParams()
    if "needs_layout_passes" in pltpu.CompilerParams.__dataclass_fields__:
        cp = dataclasses.replace(cp, needs_layout_passes=False)
    kern = pl.kernel(
        _edge_body,
        out_type=[
            jax.ShapeDtypeStruct((2, NP, D), _f32),
            jax.ShapeDtypeStruct((2, NS, D), _f32),
        ],
        mesh=mesh,
        scratch_types=(
            [pltpu.VMEM_SHARED((NP, D), _f32),
             pltpu.VMEM_SHARED((NS, D), _f32)]
            + [pltpu.VMEM((B,), _i32)] * 8
            + [pltpu.VMEM((B, D), _f32)] * 8
            + [pltpu.VMEM((H, C), _f32)]
            + [pltpu.SemaphoreType.DMA] * 6
        ),
        compiler_params=cp,
    )
    return kern(xl, xr, src, dst, att)


def _post_body(a0_ref, a1_ref, s0_ref, s1_ref, bias_ref, g_ref, b_ref,
               o_ref):
    i = pl.program_id(0)
    BR = a0_ref.shape[0]
    acc = a0_ref[...] + a1_ref[...]
    s = s0_ref[...] + s1_ref[...]
    hsel = lax.broadcasted_iota(_i32, (H, D), 0)
    csel = lax.broadcasted_iota(_i32, (H, D), 1) // C
    S = jnp.where(hsel == csel, 1.0, 0.0).astype(_f32)
    d = jnp.dot(s, S, preferred_element_type=_f32) + 1e-16
    t = acc / d + bias_ref[...]
    t = jnp.maximum(t, 0.0)
    mu = jnp.mean(t, axis=-1, keepdims=True)
    var = jnp.mean((t - mu) ** 2, axis=-1, keepdims=True)
    y = (t - mu) / jnp.sqrt(var + 1e-5) * g_ref[...] + b_ref[...]
    rows = i * BR + lax.broadcasted_iota(_i32, (BR, D), 0)
    o_ref[...] = jnp.where(rows < N, y, 0.0)


def _post(acc2, souts, bias, g, b):
    BR = 256
    sfull = souts.reshape(2, NP, H)
    return pl.pallas_call(
        _post_body,
        grid=(NP // BR,),
        in_specs=[
            pl.BlockSpec((BR, D), lambda i: (i, 0)),
            pl.BlockSpec((BR, D), lambda i: (i, 0)),
            pl.BlockSpec((BR, H), lambda i: (i, 0)),
            pl.BlockSpec((BR, H), lambda i: (i, 0)),
            pl.BlockSpec((1, D), lambda i: (0, 0)),
            pl.BlockSpec((1, D), lambda i: (0, 0)),
            pl.BlockSpec((1, D), lambda i: (0, 0)),
        ],
        out_specs=pl.BlockSpec((BR, D), lambda i: (i, 0)),
        out_shape=jax.ShapeDtypeStruct((NP, D), _f32),
    )(acc2[0], acc2[1], sfull[0], sfull[1], bias.reshape(1, D),
      g.reshape(1, D), b.reshape(1, D))


def kernel(x, edge_idx, Wl1, bl1, Wr1, br1, att1, bias1, g1, b1,
           Wl2, bl2, Wr2, br2, att2, bias2, g2, b2):
    src = jnp.concatenate([edge_idx[0], jnp.full((EPA - E,), N, _i32)])
    dst = jnp.concatenate([edge_idx[1], jnp.full((EPA - E,), N, _i32)])
    xp = jnp.concatenate([x, jnp.zeros((NP - N, D), _f32)], axis=0)

    xl1, xr1 = _proj(xp, Wl1, bl1, Wr1, br1)
    acc1, souts1 = _edge_pass(xl1, xr1, src, dst, att1)
    h = _post(acc1, souts1, bias1, g1, b1)

    xl2, xr2 = _proj(h, Wl2, bl2, Wr2, br2)
    acc2, souts2 = _edge_pass(xl2, xr2, src, dst, att2)
    h2 = _post(acc2, souts2, bias2, g2, b2)
    return h2[:N]


# speedup vs baseline: 1.0078x; 1.0078x over previous
import dataclasses
import jax
import jax.numpy as jnp
from jax import lax
from jax.experimental import pallas as pl
from jax.experimental.pallas import tpu as pltpu
from jax.experimental.pallas import tpu_sc as plsc

N = 10000
D = 128
H = 8
C = 16
NP = 10240
NS = NP // 16
E = 320000
NW = 32
B = 32
EPW = 10112
EP = NW * EPW
EPA = EP + 2 * B
NB = EPW // B
RPT = NP // 16
SPT = NS // 16

_f32 = jnp.float32
_i32 = jnp.int32


def _proj_body(x_ref, wl_ref, bl_ref, wr_ref, br_ref, xl_ref, xr_ref):
    xb = x_ref[...]
    xl_ref[...] = (
        jnp.dot(xb, wl_ref[...], preferred_element_type=_f32) + bl_ref[...]
    )
    xr_ref[...] = (
        jnp.dot(xb, wr_ref[...], preferred_element_type=_f32) + br_ref[...]
    )


def _proj(x, Wl, bl, Wr, br):
    BR = 256
    return pl.pallas_call(
        _proj_body,
        grid=(NP // BR,),
        in_specs=[
            pl.BlockSpec((BR, D), lambda i: (i, 0)),
            pl.BlockSpec((D, D), lambda i: (0, 0)),
            pl.BlockSpec((1, D), lambda i: (0, 0)),
            pl.BlockSpec((D, D), lambda i: (0, 0)),
            pl.BlockSpec((1, D), lambda i: (0, 0)),
        ],
        out_specs=[
            pl.BlockSpec((BR, D), lambda i: (i, 0)),
            pl.BlockSpec((BR, D), lambda i: (i, 0)),
        ],
        out_shape=[
            jax.ShapeDtypeStruct((NP, D), _f32),
            jax.ShapeDtypeStruct((NP, D), _f32),
        ],
    )(x, Wl, bl.reshape(1, D), Wr, br.reshape(1, D))


def _edge_body(xl_hbm, xr_hbm, src_hbm, dst_hbm, att_hbm,
               out_hbm, souts_hbm,
               acc_sh, s_sh,
               srcv0, srcv1, dstv0, dstv1, dsts0, dsts1, dst160, dst161,
               xlj0, xlj1, xri0, xri1, msg0, msg1, exm0, exm1,
               attv, semg0, semg1, sems0, sems1, semi0, semi1):
    cid = lax.axis_index("c")
    sid = lax.axis_index("s")
    wid = cid * 16 + sid

    srcv = [srcv0, srcv1]
    dstv = [dstv0, dstv1]
    dsts = [dsts0, dsts1]
    dst16 = [dst160, dst161]
    xlj = [xlj0, xlj1]
    xri = [xri0, xri1]
    msg = [msg0, msg1]
    exm = [exm0, exm1]
    semg = [semg0, semg1]
    sems = [sems0, sems1]
    semi = [semi0, semi1]

    zv = jnp.zeros((C,), _f32)
    zvi = jnp.zeros((C,), _i32)
    iota16 = lax.iota(_i32, 16)
    zero16 = jnp.zeros((16,), _f32)

    for p in range(2):
        for r in range(B):
            for j in range(D // C):
                exm[p][r, pl.ds(j * C, C)] = zv
        for g in range(B // 16):
            dsts[p][pl.ds(g * 16, 16)] = zvi
            dst16[p][pl.ds(g * 16, 16)] = zvi

    @pl.loop(0, RPT // B)
    def _zero(k):
        pltpu.sync_copy(exm0, acc_sh.at[pl.ds(sid * RPT + k * B, B)])

    pltpu.sync_copy(exm0, s_sh.at[pl.ds(sid * SPT, B)])
    pltpu.sync_copy(exm0.at[pl.ds(0, SPT - B)],
                    s_sh.at[pl.ds(sid * SPT + B, SPT - B)])

    pltpu.sync_copy(att_hbm, attv)
    att_rows = [attv[h, :] for h in range(H)]

    base0 = wid * EPW
    pltpu.async_copy(src_hbm.at[pl.ds(base0, B)], srcv[0], semi[0])
    pltpu.async_copy(dst_hbm.at[pl.ds(base0, B)], dstv[0], semi[0])
    pltpu.async_copy(src_hbm.at[pl.ds(base0 + B, B)], srcv[1], semi[1])
    pltpu.async_copy(dst_hbm.at[pl.ds(base0 + B, B)], dstv[1], semi[1])
    pltpu.make_async_copy(src_hbm.at[pl.ds(base0, B)], srcv[0],
                          semi[0]).wait()
    pltpu.make_async_copy(dst_hbm.at[pl.ds(base0, B)], dstv[0],
                          semi[0]).wait()
    pltpu.async_copy(xl_hbm.at[srcv[0]], xlj[0], semg[0])
    pltpu.async_copy(xr_hbm.at[dstv[0]], xri[0], semg[0])
    for p in range(2):
        pltpu.async_copy(exm[p], acc_sh.at[dsts[p]], sems[p], add=True)
        pltpu.async_copy(exm[p], s_sh.at[dst16[p]], sems[p], add=True)

    plsc.subcore_barrier()

    @pl.loop(0, NB // 2)
    def _iter(bj):
        for p in range(2):
            bi = 2 * bj + p
            q = 1 - p
            pltpu.make_async_copy(msg[p], acc_sh.at[dsts[p]],
                                  sems[p]).wait()
            pltpu.make_async_copy(exm[p], s_sh.at[dst16[p]],
                                  sems[p]).wait()
            for g in range(B // 16):
                dd = dsts[p][pl.ds(g * 16, 16)]
                cmb = (dd & 15) * 8
                rows = g * 16 + iota16
                for h in range(H):
                    plsc.store_scatter(exm[p], [rows, cmb + h], zero16)
            pltpu.make_async_copy(xl_hbm.at[srcv[p]], xlj[p],
                                  semg[p]).wait()
            pltpu.make_async_copy(xr_hbm.at[dstv[p]], xri[p],
                                  semg[p]).wait()
            pltpu.make_async_copy(src_hbm.at[pl.ds(0, B)], srcv[q],
                                  semi[q]).wait()
            pltpu.make_async_copy(dst_hbm.at[pl.ds(0, B)], dstv[q],
                                  semi[q]).wait()
            pltpu.async_copy(xl_hbm.at[srcv[q]], xlj[q], semg[q])
            pltpu.async_copy(xr_hbm.at[dstv[q]], xri[q], semg[q])
            rows0 = iota16
            rows1 = 16 + iota16
            d0 = dstv[p][pl.ds(0, 16)]
            d1 = dstv[p][pl.ds(16, 16)]
            dsts[p][pl.ds(0, 16)] = d0
            dsts[p][pl.ds(16, 16)] = d1
            dst16[p][pl.ds(0, 16)] = lax.shift_right_logical(d0, 4)
            dst16[p][pl.ds(16, 16)] = lax.shift_right_logical(d1, 4)
            cmb0 = (d0 & 15) * 8
            cmb1 = (d1 & 15) * 8
            for h in range(H):
                def _logit(c, lgs, _h=h):
                    lg0, lg1 = lgs
                    dcol = (c + iota16) & 15
                    col = _h * C + dcol
                    a = jnp.take(att_rows[_h], dcol)
                    l0 = plsc.load_gather(xlj[p], [rows0, col])
                    r0 = plsc.load_gather(xri[p], [rows0, col])
                    l1 = plsc.load_gather(xlj[p], [rows1, col])
                    r1 = plsc.load_gather(xri[p], [rows1, col])
                    z0 = l0 + r0
                    z0 = jnp.maximum(z0, 0.2 * z0)
                    z1 = l1 + r1
                    z1 = jnp.maximum(z1, 0.2 * z1)
                    return (lg0 + z0 * a, lg1 + z1 * a)

                zz = jnp.zeros((16,), _f32)
                lg0, lg1 = lax.fori_loop(0, C, _logit, (zz, zz),
                                         unroll=8)
                ex0 = jnp.exp(lg0)
                ex1 = jnp.exp(lg1)
                plsc.store_scatter(exm[p], [rows0, cmb0 + h], ex0)
                plsc.store_scatter(exm[p], [rows1, cmb1 + h], ex1)

                def _msg(c, t, _h=h, _ex0=ex0, _ex1=ex1):
                    dcol = (c + iota16) & 15
                    col = _h * C + dcol
                    l0 = plsc.load_gather(xlj[p], [rows0, col])
                    l1 = plsc.load_gather(xlj[p], [rows1, col])
                    plsc.store_scatter(msg[p], [rows0, col], _ex0 * l0)
                    plsc.store_scatter(msg[p], [rows1, col], _ex1 * l1)
                    return t

                lax.fori_loop(0, C, _msg, 0, unroll=8)
            pltpu.async_copy(msg[p], acc_sh.at[dsts[p]], sems[p],
                             add=True)
            pltpu.async_copy(exm[p], s_sh.at[dst16[p]], sems[p],
                             add=True)
            base2 = wid * EPW + (bi + 2) * B
            pltpu.async_copy(src_hbm.at[pl.ds(base2, B)], srcv[p],
                             semi[p])
            pltpu.async_copy(dst_hbm.at[pl.ds(base2, B)], dstv[p],
                             semi[p])

    for p in range(2):
        pltpu.make_async_copy(msg[p], acc_sh.at[dsts[p]], sems[p]).wait()
        pltpu.make_async_copy(exm[p], s_sh.at[dst16[p]], sems[p]).wait()
    pltpu.make_async_copy(xl_hbm.at[srcv[0]], xlj[0], semg[0]).wait()
    pltpu.make_async_copy(xr_hbm.at[dstv[0]], xri[0], semg[0]).wait()
    pltpu.make_async_copy(src_hbm.at[pl.ds(0, B)], srcv[1],
                          semi[1]).wait()
    pltpu.make_async_copy(dst_hbm.at[pl.ds(0, B)], dstv[1],
                          semi[1]).wait()

    plsc.subcore_barrier()

    @pl.loop(0, RPT // B)
    def _out(k):
        r0 = sid * RPT + k * B
        pltpu.sync_copy(acc_sh.at[pl.ds(r0, B)],
                        out_hbm.at[cid, pl.ds(r0, B)])

    pltpu.sync_copy(s_sh.at[pl.ds(sid * SPT, B)],
                    souts_hbm.at[cid, pl.ds(sid * SPT, B)])
    pltpu.sync_copy(s_sh.at[pl.ds(sid * SPT + B, SPT - B)],
                    souts_hbm.at[cid, pl.ds(sid * SPT + B, SPT - B)])


def _edge_pass(xl, xr, src, dst, att):
    mesh = plsc.VectorSubcoreMesh(core_axis_name="c", subcore_axis_name="s")
    cp = pltpu.CompilerParams()
    if "needs_layout_passes" in pltpu.CompilerParams.__dataclass_fields__:
        cp = dataclasses.replace(cp, needs_layout_passes=False)
    kern = pl.kernel(
        _edge_body,
        out_type=[
            jax.ShapeDtypeStruct((2, NP, D), _f32),
            jax.ShapeDtypeStruct((2, NS, D), _f32),
        ],
        mesh=mesh,
        scratch_types=(
            [pltpu.VMEM_SHARED((NP, D), _f32),
             pltpu.VMEM_SHARED((NS, D), _f32)]
            + [pltpu.VMEM((B,), _i32)] * 8
            + [pltpu.VMEM((B, D), _f32)] * 8
            + [pltpu.VMEM((H, C), _f32)]
            + [pltpu.SemaphoreType.DMA] * 6
        ),
        compiler_params=cp,
    )
    return kern(xl, xr, src, dst, att)


def _post_body(a0_ref, a1_ref, s0_ref, s1_ref, bias_ref, g_ref, b_ref,
               o_ref):
    i = pl.program_id(0)
    BR = a0_ref.shape[0]
    acc = a0_ref[...] + a1_ref[...]
    s = s0_ref[...] + s1_ref[...]
    hsel = lax.broadcasted_iota(_i32, (H, D), 0)
    csel = lax.broadcasted_iota(_i32, (H, D), 1) // C
    S = jnp.where(hsel == csel, 1.0, 0.0).astype(_f32)
    d = jnp.dot(s, S, preferred_element_type=_f32) + 1e-16
    t = acc / d + bias_ref[...]
    t = jnp.maximum(t, 0.0)
    mu = jnp.mean(t, axis=-1, keepdims=True)
    var = jnp.mean((t - mu) ** 2, axis=-1, keepdims=True)
    y = (t - mu) / jnp.sqrt(var + 1e-5) * g_ref[...] + b_ref[...]
    rows = i * BR + lax.broadcasted_iota(_i32, (BR, D), 0)
    o_ref[...] = jnp.where(rows < N, y, 0.0)


def _post(acc2, souts, bias, g, b):
    BR = 256
    sfull = souts.reshape(2, NP, H)
    return pl.pallas_call(
        _post_body,
        grid=(NP // BR,),
        in_specs=[
            pl.BlockSpec((BR, D), lambda i: (i, 0)),
            pl.BlockSpec((BR, D), lambda i: (i, 0)),
            pl.BlockSpec((BR, H), lambda i: (i, 0)),
            pl.BlockSpec((BR, H), lambda i: (i, 0)),
            pl.BlockSpec((1, D), lambda i: (0, 0)),
            pl.BlockSpec((1, D), lambda i: (0, 0)),
            pl.BlockSpec((1, D), lambda i: (0, 0)),
        ],
        out_specs=pl.BlockSpec((BR, D), lambda i: (i, 0)),
        out_shape=jax.ShapeDtypeStruct((NP, D), _f32),
    )(acc2[0], acc2[1], sfull[0], sfull[1], bias.reshape(1, D),
      g.reshape(1, D), b.reshape(1, D))


def kernel(x, edge_idx, Wl1, bl1, Wr1, br1, att1, bias1, g1, b1,
           Wl2, bl2, Wr2, br2, att2, bias2, g2, b2):
    src = jnp.concatenate([edge_idx[0], jnp.full((EPA - E,), N, _i32)])
    dst = jnp.concatenate([edge_idx[1], jnp.full((EPA - E,), N, _i32)])
    xp = jnp.concatenate([x, jnp.zeros((NP - N, D), _f32)], axis=0)

    xl1, xr1 = _proj(xp, Wl1, bl1, Wr1, br1)
    acc1, souts1 = _edge_pass(xl1, xr1, src, dst, att1)
    h = _post(acc1, souts1, bias1, g1, b1)

    xl2, xr2 = _proj(h, Wl2, bl2, Wr2, br2)
    acc2, souts2 = _edge_pass(xl2, xr2, src, dst, att2)
    h2 = _post(acc2, souts2, bias2, g2, b2)
    return h2[:N]
